# baseline (speedup 1.0000x reference)
import functools

import jax
import jax.numpy as jnp
from jax import lax
from jax.experimental import pallas as pl
from jax.experimental.pallas import tpu as pltpu
from jax.experimental.pallas import tpu_sc as plsc

K = 1024
D = 64
DP = 128
JB = 2048


def _vq_block(x_ref, w_ref, idx_ref, loss_ref):
    xb = x_ref[...]
    w = w_ref[...]
    w2 = w * (-2.0)
    mm2 = lax.dot_general(xb, w2, (((1,), (1,)), ((), ())),
                          preferred_element_type=jnp.float32)
    xsq = jnp.sum(xb * xb, axis=1, keepdims=True)
    wsq = jnp.sum(w * w, axis=1)
    d = (xsq + wsq[None, :]) + mm2
    m = jnp.min(d, axis=1, keepdims=True)
    ks = lax.broadcasted_iota(jnp.int32, d.shape, 1)
    nearest = jnp.min(jnp.where(d == m, ks, K), axis=1)
    idx_ref[...] = nearest[None, None, :]
    loss_ref[...] = jnp.broadcast_to(jnp.sum(m), (1, 1, 128))


_SC_INFO = plsc.get_sparse_core_info()
_NC, _NS = _SC_INFO.num_cores, _SC_INFO.num_subcores
_NW = _NC * _NS


def _make_sc_gather(n):
    rows_w = n // _NW
    nch = rows_w // 128
    mesh = plsc.VectorSubcoreMesh(core_axis_name="c", subcore_axis_name="s")

    @functools.partial(
        pl.kernel, mesh=mesh,
        out_type=[
            jax.ShapeDtypeStruct((n, DP), jnp.float32),
            jax.ShapeDtypeStruct((_NC, K, DP), jnp.float32),
        ],
        scratch_types=[
            pltpu.VMEM((rows_w,), jnp.int32),
            pltpu.VMEM((nch, 128), jnp.int32),
            pltpu.VMEM((128, DP), jnp.float32),
            pltpu.VMEM((128, DP), jnp.float32),
            pltpu.VMEM_SHARED((K, DP), jnp.float32),
            pltpu.SemaphoreType.DMA,
        ],
    )
    def sc_gather(wp_hbm, idx_hbm, idx8_hbm, zeros_hbm, ones_hbm, q_hbm,
                  cnt_hbm, idx_v, idx8_v, rows_v, ones_v, cnt_sh, sem):
        cid = lax.axis_index("c")
        sid = lax.axis_index("s")
        wid = sid * _NC + cid
        base = wid * rows_w
        pltpu.sync_copy(idx_hbm.at[wid], idx_v)
        pltpu.sync_copy(idx8_hbm.at[wid], idx8_v)
        pltpu.sync_copy(ones_hbm, ones_v)

        @pl.when(sid == 0)
        def _init_counts():
            pltpu.sync_copy(zeros_hbm, cnt_sh)

        for j in range(nch):
            pltpu.async_copy(
                wp_hbm.at[idx_v.at[pl.ds(j * 128, 128)]], rows_v, sem).wait()
            pltpu.sync_copy(rows_v, q_hbm.at[pl.ds(base + j * 128, 128)])

        plsc.subcore_barrier()
        for j in range(nch):
            pltpu.sync_copy(ones_v, cnt_sh.at[idx8_v.at[j]], add=True)
        plsc.subcore_barrier()

        @pl.when(sid == 0)
        def _write_counts():
            pltpu.sync_copy(cnt_sh, cnt_hbm.at[cid])

    return sc_gather


def kernel(inputs, W, beta):
    B, C, H, Wd = inputs.shape
    N = B * H * Wd
    nb = N // JB
    x = jnp.transpose(inputs, (0, 2, 3, 1)).reshape(N, D)
    idx, lsum = pl.pallas_call(
        _vq_block,
        grid=(nb,),
        in_specs=[
            pl.BlockSpec((JB, D), lambda j: (j, 0)),
            pl.BlockSpec((K, D), lambda j: (0, 0)),
        ],
        out_specs=[
            pl.BlockSpec((1, 1, JB), lambda j: (j, 0, 0)),
            pl.BlockSpec((1, 1, 128), lambda j: (j, 0, 0)),
        ],
        out_shape=[
            jax.ShapeDtypeStruct((nb, 1, JB), jnp.int32),
            jax.ShapeDtypeStruct((nb, 1, 128), jnp.float32),
        ],
    )(x, W)
    wp = jnp.pad(W, ((0, 0), (0, DP - D)))
    rows_w = N // _NW
    idx2 = idx.reshape(_NW, rows_w)
    idx3 = idx.reshape(_NW, rows_w // 128, 128)
    zeros = jnp.zeros((K, DP), jnp.float32)
    ones = jnp.ones((128, DP), jnp.float32)
    q, cnt = _make_sc_gather(N)(wp, idx2, idx3, zeros, ones)
    loss_mean = jnp.sum(lsum[:, 0, 0]) / (N * D)
    loss = loss_mean + beta * loss_mean
    e_mean = jnp.sum(cnt[:, :, 0], axis=0) / N
    perplexity = jnp.exp(-jnp.sum(e_mean * jnp.log(e_mean + 1e-10)))
    quantized_out = jnp.transpose(q[:, :D].reshape(B, H, Wd, C), (0, 3, 1, 2))
    return (loss, quantized_out, perplexity)


# speedup vs baseline: 1.4136x; 1.4136x over previous
import functools

import jax
import jax.numpy as jnp
from jax import lax
from jax.experimental import pallas as pl
from jax.experimental.pallas import tpu as pltpu

K = 1024
D = 64
JB = 4096


def _vq_block(x_ref, w_ref, q_ref, loss_ref, cnt_ref):
    xb = x_ref[...]
    w = w_ref[...]
    w2 = w * (-2.0)
    mm2 = lax.dot_general(xb, w2, (((1,), (1,)), ((), ())),
                          preferred_element_type=jnp.float32)
    xsq = jnp.sum(xb * xb, axis=1, keepdims=True)
    wsq = jnp.sum(w * w, axis=1)
    d = (xsq + wsq[None, :]) + mm2
    m = jnp.min(d, axis=1, keepdims=True)
    ks = lax.broadcasted_iota(jnp.int32, d.shape, 1)
    nearest = jnp.min(jnp.where(d == m, ks, K), axis=1)
    oh = (ks == nearest[:, None]).astype(jnp.float32)
    qc = lax.dot_general(w, oh, (((0,), (1,)), ((), ())),
                         preferred_element_type=jnp.float32)
    for i in range(JB // 1024):
        q_ref[i] = qc[:, i * 1024:(i + 1) * 1024]
    loss_ref[...] = jnp.broadcast_to(jnp.sum(m), (1, 1, 128))
    cnt_ref[...] = jnp.sum(oh, axis=0)[None, None, :]


def kernel(inputs, W, beta):
    B, C, H, Wd = inputs.shape
    N = B * H * Wd
    nb = N // JB
    x = jnp.transpose(inputs, (0, 2, 3, 1)).reshape(N, D)
    q, lsum, cnt = pl.pallas_call(
        _vq_block,
        grid=(nb,),
        in_specs=[
            pl.BlockSpec((JB, D), lambda j: (j, 0)),
            pl.BlockSpec((K, D), lambda j: (0, 0)),
        ],
        out_specs=[
            pl.BlockSpec((JB // 1024, D, H * Wd), lambda j: (j, 0, 0)),
            pl.BlockSpec((1, 1, 128), lambda j: (j, 0, 0)),
            pl.BlockSpec((1, 1, K), lambda j: (j, 0, 0)),
        ],
        out_shape=[
            jax.ShapeDtypeStruct((B, D, H * Wd), jnp.float32),
            jax.ShapeDtypeStruct((nb, 1, 128), jnp.float32),
            jax.ShapeDtypeStruct((nb, 1, K), jnp.float32),
        ],
    )(x, W)
    loss_mean = jnp.sum(lsum[:, 0, 0]) / (N * D)
    loss = loss_mean + beta * loss_mean
    e_mean = jnp.sum(cnt[:, 0, :], axis=0) / N
    perplexity = jnp.exp(-jnp.sum(e_mean * jnp.log(e_mean + 1e-10)))
    quantized_out = q.reshape(B, C, H, Wd)
    return (loss, quantized_out, perplexity)
